# initial kernel scaffold (unmeasured)
import jax
import jax.numpy as jnp
from jax import lax
from jax.experimental import pallas as pl
from jax.experimental.pallas import tpu as pltpu

N_DEV = 16
S = 4096
D = 1024
DH = 128
H_LOC = 8
CHUNK = S // N_DEV
BQ = 512
SCALE = 0.08838834764831843
EPS = 1e-5

_CompilerParams = getattr(pltpu, "CompilerParams", None) or getattr(
    pltpu, "TPUCompilerParams"
)


def _ln_mod(xb, scale_row, shift_row):
    m = jnp.mean(xb, axis=-1, keepdims=True)
    c = xb - m
    v = jnp.mean(c * c, axis=-1, keepdims=True)
    xn = c * lax.rsqrt(v + EPS)
    return xn * (1.0 + scale_row) + shift_row


def _qkv_body(x_ref, sa_ref, sha_ref, w_ref, o_ref):
    xm = _ln_mod(x_ref[...], sa_ref[0, :], sha_ref[0, :])
    o_ref[...] = jnp.dot(
        xm.astype(jnp.bfloat16), w_ref[...], preferred_element_type=jnp.float32
    ).astype(jnp.bfloat16)


def _qkv(x2, sa, sha, wqkv):
    return pl.pallas_call(
        _qkv_body,
        grid=(S // BQ,),
        in_specs=[
            pl.BlockSpec((BQ, D), lambda i: (i, 0)),
            pl.BlockSpec((1, D), lambda i: (0, 0)),
            pl.BlockSpec((1, D), lambda i: (0, 0)),
            pl.BlockSpec((D, 3 * D), lambda i: (0, 0)),
        ],
        out_specs=pl.BlockSpec((BQ, 3 * D), lambda i: (i, 0)),
        out_shape=jax.ShapeDtypeStruct((S, 3 * D), jnp.bfloat16),
    )(x2, sa, sha, wqkv)


def _attn_body(q_ref, k_ref, v_ref, o_ref):
    s = (
        lax.dot_general(
            q_ref[...],
            k_ref[...],
            (((1,), (1,)), ((), ())),
            preferred_element_type=jnp.float32,
        )
        * SCALE
    )
    m = jnp.max(s, axis=-1, keepdims=True)
    p = jnp.exp(s - m)
    l = jnp.sum(p, axis=-1, keepdims=True)
    o = jnp.dot(
        p.astype(jnp.bfloat16), v_ref[...], preferred_element_type=jnp.float32
    )
    o_ref[...] = (o / l).astype(jnp.bfloat16)


def _attention(qkv):
    return pl.pallas_call(
        _attn_body,
        grid=(H_LOC, S // BQ),
        in_specs=[
            pl.BlockSpec((BQ, DH), lambda h, i: (i, h)),
            pl.BlockSpec((S, DH), lambda h, i: (0, H_LOC + h)),
            pl.BlockSpec((S, DH), lambda h, i: (0, 2 * H_LOC + h)),
        ],
        out_specs=pl.BlockSpec((BQ, DH), lambda h, i: (i, h)),
        out_shape=jax.ShapeDtypeStruct((S, D), jnp.bfloat16),
    )(qkv, qkv, qkv)


def _matmul_body(a_ref, w_ref, o_ref):
    o_ref[...] = jnp.dot(
        a_ref[...], w_ref[...], preferred_element_type=jnp.float32
    )


def _matmul(a, w):
    k = a.shape[1]
    return pl.pallas_call(
        _matmul_body,
        grid=(S // BQ,),
        in_specs=[
            pl.BlockSpec((BQ, k), lambda i: (i, 0)),
            pl.BlockSpec((k, D), lambda i: (0, 0)),
        ],
        out_specs=pl.BlockSpec((BQ, D), lambda i: (i, 0)),
        out_shape=jax.ShapeDtypeStruct((S, D), jnp.float32),
    )(a, w)


def _ff_body(x_ref, sm_ref, shm_ref, w1_ref, w2_ref, o_ref):
    xm = _ln_mod(x_ref[...], sm_ref[0, :], shm_ref[0, :]).astype(jnp.bfloat16)
    h = jnp.dot(xm, w1_ref[...], preferred_element_type=jnp.float32)
    h = h * jax.nn.sigmoid(h)
    o_ref[...] = jnp.dot(
        h.astype(jnp.bfloat16), w2_ref[...], preferred_element_type=jnp.float32
    )


def _ffn(x1, sm, shm, w1, w2):
    dff = w1.shape[1]
    return pl.pallas_call(
        _ff_body,
        grid=(S // BQ,),
        in_specs=[
            pl.BlockSpec((BQ, D), lambda i: (i, 0)),
            pl.BlockSpec((1, D), lambda i: (0, 0)),
            pl.BlockSpec((1, D), lambda i: (0, 0)),
            pl.BlockSpec((D, dff), lambda i: (0, 0)),
            pl.BlockSpec((dff, D), lambda i: (0, 0)),
        ],
        out_specs=pl.BlockSpec((BQ, D), lambda i: (i, 0)),
        out_shape=jax.ShapeDtypeStruct((S, D), jnp.float32),
    )(x1, sm, shm, w1, w2)


def _ar_body(partial_ref, res_ref, gate_ref, out_ref, comm_ref, send_sems,
             recv_sems, credit_sem):
    my = lax.axis_index("i")
    left = lax.rem(my - 1 + N_DEV, N_DEV)
    right = lax.rem(my + 1, N_DEV)
    gate = gate_ref[0, :]

    barrier = pltpu.get_barrier_semaphore()
    for nbr in (left, right):
        pl.semaphore_signal(
            barrier, inc=1, device_id=(nbr,),
            device_id_type=pl.DeviceIdType.MESH,
        )
    pl.semaphore_wait(barrier, 2)

    def rows(idx):
        return pl.ds(idx * CHUNK, CHUNK)

    comm_ref[0, :, :] = partial_ref[rows(my), :]

    for s in range(2 * N_DEV - 2):
        src_slot = s % 2
        dst_slot = (s + 1) % 2
        if s > 0:
            pl.semaphore_wait(credit_sem, 1)
        rdma = pltpu.make_async_remote_copy(
            src_ref=comm_ref.at[src_slot],
            dst_ref=comm_ref.at[dst_slot],
            send_sem=send_sems.at[src_slot],
            recv_sem=recv_sems.at[dst_slot],
            device_id=(right,),
            device_id_type=pl.DeviceIdType.MESH,
        )
        rdma.start()
        rdma.wait_send()
        rdma.wait_recv()
        if s < N_DEV - 1:
            cidx = lax.rem(my - s - 1 + 2 * N_DEV, N_DEV)
            acc = comm_ref[dst_slot, :, :] + partial_ref[rows(cidx), :]
            comm_ref[dst_slot, :, :] = acc
            if s == N_DEV - 2:
                out_ref[rows(cidx), :] = res_ref[rows(cidx), :] + gate * acc
        else:
            t = s - (N_DEV - 1)
            cidx = lax.rem(my - t + 2 * N_DEV, N_DEV)
            out_ref[rows(cidx), :] = (
                res_ref[rows(cidx), :] + gate * comm_ref[dst_slot, :, :]
            )
        if s < 2 * N_DEV - 3:
            pl.semaphore_signal(
                credit_sem, inc=1, device_id=(left,),
                device_id_type=pl.DeviceIdType.MESH,
            )


def _allreduce_residual(partial, res, gate, collective_id):
    return pl.pallas_call(
        _ar_body,
        in_specs=[
            pl.BlockSpec(memory_space=pltpu.VMEM),
            pl.BlockSpec(memory_space=pltpu.VMEM),
            pl.BlockSpec(memory_space=pltpu.VMEM),
        ],
        out_specs=pl.BlockSpec(memory_space=pltpu.VMEM),
        out_shape=jax.ShapeDtypeStruct((S, D), jnp.float32),
        scratch_shapes=[
            pltpu.VMEM((2, CHUNK, D), jnp.float32),
            pltpu.SemaphoreType.DMA((2,)),
            pltpu.SemaphoreType.DMA((2,)),
            pltpu.SemaphoreType.REGULAR,
        ],
        compiler_params=_CompilerParams(collective_id=collective_id),
    )(partial, res, gate)


def kernel(x, Wq, Wk, Wv, Wo, t_emb, W_mod, W_ff1, W_ff2):
    x2 = x.reshape(S, D)
    mod = t_emb @ W_mod
    sa, sha, ga, sm, shm, gm = jnp.split(mod, 6, axis=-1)

    wqkv = jnp.concatenate([Wq, Wk, Wv], axis=1).astype(jnp.bfloat16)
    qkv = _qkv(x2, sa, sha, wqkv)
    attn = _attention(qkv)
    part1 = _matmul(attn, Wo.astype(jnp.bfloat16))
    x1 = _allreduce_residual(part1, x2, ga, collective_id=0)

    part2 = _ffn(x1, sm, shm, W_ff1.astype(jnp.bfloat16),
                 W_ff2.astype(jnp.bfloat16))
    out = _allreduce_residual(part2, x1, gm, collective_id=1)
    return out.reshape(1, S, D)


# baseline (device time: 1476101 ns/iter reference)
import jax
import jax.numpy as jnp
from jax import lax
from jax.experimental import pallas as pl
from jax.experimental.pallas import tpu as pltpu

N_DEV = 16
S = 4096
D = 1024
DH = 128
H_LOC = 8
CHUNK = S // N_DEV
BQ = 512
SCALE = 0.08838834764831843
EPS = 1e-5

_CompilerParams = getattr(pltpu, "CompilerParams", None) or getattr(
    pltpu, "TPUCompilerParams"
)


def _ln_mod(xb, scale_row, shift_row):
    m = jnp.mean(xb, axis=-1, keepdims=True)
    c = xb - m
    v = jnp.mean(c * c, axis=-1, keepdims=True)
    xn = c * lax.rsqrt(v + EPS)
    return xn * (1.0 + scale_row) + shift_row


def _qkv_body(x_ref, sa_ref, sha_ref, w_ref, o_ref):
    xm = _ln_mod(x_ref[...], sa_ref[0, :], sha_ref[0, :])
    o_ref[...] = jnp.dot(
        xm.astype(jnp.bfloat16), w_ref[...], preferred_element_type=jnp.float32
    ).astype(jnp.bfloat16)


def _qkv(x2, sa, sha, wqkv):
    return pl.pallas_call(
        _qkv_body,
        grid=(S // BQ,),
        in_specs=[
            pl.BlockSpec((BQ, D), lambda i: (i, 0)),
            pl.BlockSpec((1, D), lambda i: (0, 0)),
            pl.BlockSpec((1, D), lambda i: (0, 0)),
            pl.BlockSpec((D, 3 * D), lambda i: (0, 0)),
        ],
        out_specs=pl.BlockSpec((BQ, 3 * D), lambda i: (i, 0)),
        out_shape=jax.ShapeDtypeStruct((S, 3 * D), jnp.bfloat16),
    )(x2, sa, sha, wqkv)


def _attn_body(q_ref, k_ref, v_ref, o_ref):
    s = (
        lax.dot_general(
            q_ref[...],
            k_ref[...],
            (((1,), (1,)), ((), ())),
            preferred_element_type=jnp.float32,
        )
        * SCALE
    )
    m = jnp.max(s, axis=-1, keepdims=True)
    p = jnp.exp(s - m)
    l = jnp.sum(p, axis=-1, keepdims=True)
    o = jnp.dot(
        p.astype(jnp.bfloat16), v_ref[...], preferred_element_type=jnp.float32
    )
    o_ref[...] = (o / l).astype(jnp.bfloat16)


def _attention(qkv):
    return pl.pallas_call(
        _attn_body,
        grid=(H_LOC, S // BQ),
        in_specs=[
            pl.BlockSpec((BQ, DH), lambda h, i: (i, h)),
            pl.BlockSpec((S, DH), lambda h, i: (0, H_LOC + h)),
            pl.BlockSpec((S, DH), lambda h, i: (0, 2 * H_LOC + h)),
        ],
        out_specs=pl.BlockSpec((BQ, DH), lambda h, i: (i, h)),
        out_shape=jax.ShapeDtypeStruct((S, D), jnp.bfloat16),
    )(qkv, qkv, qkv)


def _matmul_body(a_ref, w_ref, o_ref):
    o_ref[...] = jnp.dot(
        a_ref[...], w_ref[...], preferred_element_type=jnp.float32
    )


def _matmul(a, w):
    k = a.shape[1]
    return pl.pallas_call(
        _matmul_body,
        grid=(S // BQ,),
        in_specs=[
            pl.BlockSpec((BQ, k), lambda i: (i, 0)),
            pl.BlockSpec((k, D), lambda i: (0, 0)),
        ],
        out_specs=pl.BlockSpec((BQ, D), lambda i: (i, 0)),
        out_shape=jax.ShapeDtypeStruct((S, D), jnp.float32),
    )(a, w)


def _ff_body(x_ref, sm_ref, shm_ref, w1_ref, w2_ref, o_ref):
    xm = _ln_mod(x_ref[...], sm_ref[0, :], shm_ref[0, :]).astype(jnp.bfloat16)
    h = jnp.dot(xm, w1_ref[...], preferred_element_type=jnp.float32)
    h = h * jax.nn.sigmoid(h)
    o_ref[...] = jnp.dot(
        h.astype(jnp.bfloat16), w2_ref[...], preferred_element_type=jnp.float32
    )


def _ffn(x1, sm, shm, w1, w2):
    dff = w1.shape[1]
    return pl.pallas_call(
        _ff_body,
        grid=(S // BQ,),
        in_specs=[
            pl.BlockSpec((BQ, D), lambda i: (i, 0)),
            pl.BlockSpec((1, D), lambda i: (0, 0)),
            pl.BlockSpec((1, D), lambda i: (0, 0)),
            pl.BlockSpec((D, dff), lambda i: (0, 0)),
            pl.BlockSpec((dff, D), lambda i: (0, 0)),
        ],
        out_specs=pl.BlockSpec((BQ, D), lambda i: (i, 0)),
        out_shape=jax.ShapeDtypeStruct((S, D), jnp.float32),
    )(x1, sm, shm, w1, w2)


def _ar_body(partial_hbm, res_hbm, gate_ref, out_hbm, comm_ref, pbuf, rbuf,
             send_sems, recv_sems, credit_sem, local_sem):
    my = lax.axis_index("i")
    left = lax.rem(my - 1 + N_DEV, N_DEV)
    right = lax.rem(my + 1, N_DEV)
    gate = gate_ref[0, :]

    barrier = pltpu.get_barrier_semaphore()
    for nbr in (left, right):
        pl.semaphore_signal(
            barrier, inc=1, device_id=(nbr,),
            device_id_type=pl.DeviceIdType.MESH,
        )
    pl.semaphore_wait(barrier, 2)

    def rows(idx):
        return pl.ds(idx * CHUNK, CHUNK)

    def copy(src, dst):
        cp = pltpu.make_async_copy(src, dst, local_sem)
        cp.start()
        cp.wait()

    copy(partial_hbm.at[rows(my)], comm_ref.at[0])

    def store_out(cidx, summed):
        rbuf[...] = rbuf[...] + gate * summed
        copy(rbuf, out_hbm.at[rows(cidx)])

    for s in range(2 * N_DEV - 2):
        src_slot = s % 2
        dst_slot = (s + 1) % 2
        if s > 0:
            pl.semaphore_wait(credit_sem, 1)
        rdma = pltpu.make_async_remote_copy(
            src_ref=comm_ref.at[src_slot],
            dst_ref=comm_ref.at[dst_slot],
            send_sem=send_sems.at[src_slot],
            recv_sem=recv_sems.at[dst_slot],
            device_id=(right,),
            device_id_type=pl.DeviceIdType.MESH,
        )
        rdma.start()
        if s < N_DEV - 1:
            cidx = lax.rem(my - s - 1 + 2 * N_DEV, N_DEV)
            copy(partial_hbm.at[rows(cidx)], pbuf)
            if s == N_DEV - 2:
                copy(res_hbm.at[rows(cidx)], rbuf)
            rdma.wait_send()
            rdma.wait_recv()
            acc = comm_ref[dst_slot, :, :] + pbuf[...]
            comm_ref[dst_slot, :, :] = acc
            if s == N_DEV - 2:
                store_out(cidx, acc)
        else:
            t = s - (N_DEV - 1)
            cidx = lax.rem(my - t + 2 * N_DEV, N_DEV)
            copy(res_hbm.at[rows(cidx)], rbuf)
            rdma.wait_send()
            rdma.wait_recv()
            store_out(cidx, comm_ref[dst_slot, :, :])
        if s < 2 * N_DEV - 3:
            pl.semaphore_signal(
                credit_sem, inc=1, device_id=(left,),
                device_id_type=pl.DeviceIdType.MESH,
            )


def _allreduce_residual(partial, res, gate, collective_id):
    return pl.pallas_call(
        _ar_body,
        in_specs=[
            pl.BlockSpec(memory_space=pl.ANY),
            pl.BlockSpec(memory_space=pl.ANY),
            pl.BlockSpec(memory_space=pltpu.VMEM),
        ],
        out_specs=pl.BlockSpec(memory_space=pl.ANY),
        out_shape=jax.ShapeDtypeStruct((S, D), jnp.float32),
        scratch_shapes=[
            pltpu.VMEM((2, CHUNK, D), jnp.float32),
            pltpu.VMEM((CHUNK, D), jnp.float32),
            pltpu.VMEM((CHUNK, D), jnp.float32),
            pltpu.SemaphoreType.DMA((2,)),
            pltpu.SemaphoreType.DMA((2,)),
            pltpu.SemaphoreType.REGULAR,
            pltpu.SemaphoreType.DMA,
        ],
        compiler_params=_CompilerParams(collective_id=collective_id),
    )(partial, res, gate)


def kernel(x, Wq, Wk, Wv, Wo, t_emb, W_mod, W_ff1, W_ff2):
    x2 = x.reshape(S, D)
    mod = t_emb @ W_mod
    sa, sha, ga, sm, shm, gm = jnp.split(mod, 6, axis=-1)

    wqkv = jnp.concatenate([Wq, Wk, Wv], axis=1).astype(jnp.bfloat16)
    qkv = _qkv(x2, sa, sha, wqkv)
    attn = _attention(qkv)
    part1 = _matmul(attn, Wo.astype(jnp.bfloat16))
    x1 = _allreduce_residual(part1, x2, ga, collective_id=0)

    part2 = _ffn(x1, sm, shm, W_ff1.astype(jnp.bfloat16),
                 W_ff2.astype(jnp.bfloat16))
    out = _allreduce_residual(part2, x1, gm, collective_id=1)
    return out.reshape(1, S, D)


# device time: 1011148 ns/iter; 1.4598x vs baseline; 1.4598x over previous
import jax
import jax.numpy as jnp
from jax import lax
from jax.experimental import pallas as pl
from jax.experimental.pallas import tpu as pltpu

N_DEV = 16
S = 4096
D = 1024
DH = 128
H_LOC = 8
CHUNK = S // N_DEV
BQ = 512
SCALE = 0.08838834764831843
EPS = 1e-5

_CompilerParams = getattr(pltpu, "CompilerParams", None) or getattr(
    pltpu, "TPUCompilerParams"
)


def _ln_mod(xb, scale_row, shift_row):
    m = jnp.mean(xb, axis=-1, keepdims=True)
    c = xb - m
    v = jnp.mean(c * c, axis=-1, keepdims=True)
    xn = c * lax.rsqrt(v + EPS)
    return xn * (1.0 + scale_row) + shift_row


def _qkv_body(x_ref, sa_ref, sha_ref, w_ref, o_ref):
    xm = _ln_mod(x_ref[...], sa_ref[0, :], sha_ref[0, :])
    o_ref[...] = jnp.dot(
        xm.astype(jnp.bfloat16), w_ref[...], preferred_element_type=jnp.float32
    ).astype(jnp.bfloat16)


def _qkv(x2, sa, sha, wqkv):
    return pl.pallas_call(
        _qkv_body,
        grid=(S // BQ,),
        in_specs=[
            pl.BlockSpec((BQ, D), lambda i: (i, 0)),
            pl.BlockSpec((1, D), lambda i: (0, 0)),
            pl.BlockSpec((1, D), lambda i: (0, 0)),
            pl.BlockSpec((D, 3 * D), lambda i: (0, 0)),
        ],
        out_specs=pl.BlockSpec((BQ, 3 * D), lambda i: (i, 0)),
        out_shape=jax.ShapeDtypeStruct((S, 3 * D), jnp.bfloat16),
    )(x2, sa, sha, wqkv)


def _attn_body(q_ref, k_ref, v_ref, o_ref):
    s = (
        lax.dot_general(
            q_ref[...],
            k_ref[...],
            (((1,), (1,)), ((), ())),
            preferred_element_type=jnp.float32,
        )
        * SCALE
    )
    m = jnp.max(s, axis=-1, keepdims=True)
    p = jnp.exp(s - m)
    l = jnp.sum(p, axis=-1, keepdims=True)
    o = jnp.dot(
        p.astype(jnp.bfloat16), v_ref[...], preferred_element_type=jnp.float32
    )
    o_ref[...] = (o / l).astype(jnp.bfloat16)


def _attention(qkv):
    return pl.pallas_call(
        _attn_body,
        grid=(H_LOC, S // BQ),
        in_specs=[
            pl.BlockSpec((BQ, DH), lambda h, i: (i, h)),
            pl.BlockSpec((S, DH), lambda h, i: (0, H_LOC + h)),
            pl.BlockSpec((S, DH), lambda h, i: (0, 2 * H_LOC + h)),
        ],
        out_specs=pl.BlockSpec((BQ, DH), lambda h, i: (i, h)),
        out_shape=jax.ShapeDtypeStruct((S, D), jnp.bfloat16),
    )(qkv, qkv, qkv)


def _matmul_body(a_ref, w_ref, o_ref):
    o_ref[...] = jnp.dot(
        a_ref[...], w_ref[...], preferred_element_type=jnp.float32
    ).astype(jnp.bfloat16)


def _matmul(a, w):
    k = a.shape[1]
    return pl.pallas_call(
        _matmul_body,
        grid=(S // BQ,),
        in_specs=[
            pl.BlockSpec((BQ, k), lambda i: (i, 0)),
            pl.BlockSpec((k, D), lambda i: (0, 0)),
        ],
        out_specs=pl.BlockSpec((BQ, D), lambda i: (i, 0)),
        out_shape=jax.ShapeDtypeStruct((S, D), jnp.bfloat16),
    )(a, w)


def _ff_body(x_ref, sm_ref, shm_ref, w1_ref, w2_ref, o_ref):
    xm = _ln_mod(x_ref[...], sm_ref[0, :], shm_ref[0, :]).astype(jnp.bfloat16)
    h = jnp.dot(xm, w1_ref[...], preferred_element_type=jnp.float32)
    h = h * jax.nn.sigmoid(h)
    o_ref[...] = jnp.dot(
        h.astype(jnp.bfloat16), w2_ref[...], preferred_element_type=jnp.float32
    ).astype(jnp.bfloat16)


def _ffn(x1, sm, shm, w1, w2):
    dff = w1.shape[1]
    return pl.pallas_call(
        _ff_body,
        grid=(S // BQ,),
        in_specs=[
            pl.BlockSpec((BQ, D), lambda i: (i, 0)),
            pl.BlockSpec((1, D), lambda i: (0, 0)),
            pl.BlockSpec((1, D), lambda i: (0, 0)),
            pl.BlockSpec((D, dff), lambda i: (0, 0)),
            pl.BlockSpec((dff, D), lambda i: (0, 0)),
        ],
        out_specs=pl.BlockSpec((BQ, D), lambda i: (i, 0)),
        out_shape=jax.ShapeDtypeStruct((S, D), jnp.bfloat16),
    )(x1, sm, shm, w1, w2)


CH2 = S // (2 * N_DEV)


def _ar_body(partial_hbm, res_hbm, gate_ref, out_hbm,
             commA, commB, pbufA, pbufB, rbufA, rbufB,
             sendA, recvA, sendB, recvB, creditA, creditB, local_sem):
    my = lax.axis_index("i")
    left = lax.rem(my - 1 + N_DEV, N_DEV)
    right = lax.rem(my + 1, N_DEV)
    gate = gate_ref[0, :]
    f32 = jnp.float32

    barrier = pltpu.get_barrier_semaphore()
    for nbr in (left, right):
        pl.semaphore_signal(
            barrier, inc=1, device_id=(nbr,),
            device_id_type=pl.DeviceIdType.MESH,
        )
    pl.semaphore_wait(barrier, 2)

    def rowsA(idx):
        return pl.ds(idx * CH2, CH2)

    def rowsB(idx):
        return pl.ds(S // 2 + idx * CH2, CH2)

    def copy(src, dst):
        cp = pltpu.make_async_copy(src, dst, local_sem)
        cp.start()
        cp.wait()

    copy(partial_hbm.at[rowsA(my)], commA.at[0])
    copy(partial_hbm.at[rowsB(my)], commB.at[0])

    def store_out(rbuf, rows, cidx, summed_f32):
        rbuf[...] = rbuf[...] + gate * summed_f32
        copy(rbuf, out_hbm.at[rows(cidx)])

    for s in range(2 * N_DEV - 2):
        src_slot = s % 2
        dst_slot = (s + 1) % 2
        if s > 0:
            pl.semaphore_wait(creditA, 1)
            pl.semaphore_wait(creditB, 1)
        rdmaA = pltpu.make_async_remote_copy(
            src_ref=commA.at[src_slot],
            dst_ref=commA.at[dst_slot],
            send_sem=sendA.at[src_slot],
            recv_sem=recvA.at[dst_slot],
            device_id=(right,),
            device_id_type=pl.DeviceIdType.MESH,
        )
        rdmaB = pltpu.make_async_remote_copy(
            src_ref=commB.at[src_slot],
            dst_ref=commB.at[dst_slot],
            send_sem=sendB.at[src_slot],
            recv_sem=recvB.at[dst_slot],
            device_id=(left,),
            device_id_type=pl.DeviceIdType.MESH,
        )
        rdmaA.start()
        rdmaB.start()
        if s < N_DEV - 1:
            ciA = lax.rem(my - s - 1 + 2 * N_DEV, N_DEV)
            ciB = lax.rem(my + s + 1, N_DEV)
            copy(partial_hbm.at[rowsA(ciA)], pbufA)
            copy(partial_hbm.at[rowsB(ciB)], pbufB)
            if s == N_DEV - 2:
                copy(res_hbm.at[rowsA(ciA)], rbufA)
                copy(res_hbm.at[rowsB(ciB)], rbufB)
            rdmaA.wait_send()
            rdmaA.wait_recv()
            accA = commA[dst_slot, :, :].astype(f32) + pbufA[...].astype(f32)
            commA[dst_slot, :, :] = accA.astype(jnp.bfloat16)
            rdmaB.wait_send()
            rdmaB.wait_recv()
            accB = commB[dst_slot, :, :].astype(f32) + pbufB[...].astype(f32)
            commB[dst_slot, :, :] = accB.astype(jnp.bfloat16)
            if s == N_DEV - 2:
                store_out(rbufA, rowsA, ciA, accA)
                store_out(rbufB, rowsB, ciB, accB)
        else:
            t = s - (N_DEV - 1)
            ciA = lax.rem(my - t + 2 * N_DEV, N_DEV)
            ciB = lax.rem(my + t, N_DEV)
            copy(res_hbm.at[rowsA(ciA)], rbufA)
            copy(res_hbm.at[rowsB(ciB)], rbufB)
            rdmaA.wait_send()
            rdmaA.wait_recv()
            store_out(rbufA, rowsA, ciA, commA[dst_slot, :, :].astype(f32))
            rdmaB.wait_send()
            rdmaB.wait_recv()
            store_out(rbufB, rowsB, ciB, commB[dst_slot, :, :].astype(f32))
        if s < 2 * N_DEV - 3:
            pl.semaphore_signal(
                creditA, inc=1, device_id=(left,),
                device_id_type=pl.DeviceIdType.MESH,
            )
            pl.semaphore_signal(
                creditB, inc=1, device_id=(right,),
                device_id_type=pl.DeviceIdType.MESH,
            )


def _allreduce_residual(partial, res, gate, collective_id):
    return pl.pallas_call(
        _ar_body,
        in_specs=[
            pl.BlockSpec(memory_space=pl.ANY),
            pl.BlockSpec(memory_space=pl.ANY),
            pl.BlockSpec(memory_space=pltpu.VMEM),
        ],
        out_specs=pl.BlockSpec(memory_space=pl.ANY),
        out_shape=jax.ShapeDtypeStruct((S, D), jnp.float32),
        scratch_shapes=[
            pltpu.VMEM((2, CH2, D), jnp.bfloat16),
            pltpu.VMEM((2, CH2, D), jnp.bfloat16),
            pltpu.VMEM((CH2, D), jnp.bfloat16),
            pltpu.VMEM((CH2, D), jnp.bfloat16),
            pltpu.VMEM((CH2, D), jnp.float32),
            pltpu.VMEM((CH2, D), jnp.float32),
            pltpu.SemaphoreType.DMA((2,)),
            pltpu.SemaphoreType.DMA((2,)),
            pltpu.SemaphoreType.DMA((2,)),
            pltpu.SemaphoreType.DMA((2,)),
            pltpu.SemaphoreType.REGULAR,
            pltpu.SemaphoreType.REGULAR,
            pltpu.SemaphoreType.DMA,
        ],
        compiler_params=_CompilerParams(collective_id=collective_id),
    )(partial, res, gate)


def kernel(x, Wq, Wk, Wv, Wo, t_emb, W_mod, W_ff1, W_ff2):
    x2 = x.reshape(S, D)
    mod = t_emb @ W_mod
    sa, sha, ga, sm, shm, gm = jnp.split(mod, 6, axis=-1)

    wqkv = jnp.concatenate([Wq, Wk, Wv], axis=1).astype(jnp.bfloat16)
    qkv = _qkv(x2, sa, sha, wqkv)
    attn = _attention(qkv)
    part1 = _matmul(attn, Wo.astype(jnp.bfloat16))
    x1 = _allreduce_residual(part1, x2, ga, collective_id=0)

    part2 = _ffn(x1, sm, shm, W_ff1.astype(jnp.bfloat16),
                 W_ff2.astype(jnp.bfloat16))
    out = _allreduce_residual(part2, x1, gm, collective_id=1)
    return out.reshape(1, S, D)


# device time: 804336 ns/iter; 1.8352x vs baseline; 1.2571x over previous
import jax
import jax.numpy as jnp
from jax import lax
from jax.experimental import pallas as pl
from jax.experimental.pallas import tpu as pltpu

N_DEV = 16
S = 4096
D = 1024
DH = 128
H_LOC = 8
CHUNK = S // N_DEV
BQ = 512
SCALE = 0.08838834764831843
EPS = 1e-5

_CompilerParams = getattr(pltpu, "CompilerParams", None) or getattr(
    pltpu, "TPUCompilerParams"
)


def _ln_mod(xb, scale_row, shift_row):
    m = jnp.mean(xb, axis=-1, keepdims=True)
    c = xb - m
    v = jnp.mean(c * c, axis=-1, keepdims=True)
    xn = c * lax.rsqrt(v + EPS)
    return xn * (1.0 + scale_row) + shift_row


def _qkv_body(x_ref, sa_ref, sha_ref, w_ref, o_ref):
    xm = _ln_mod(x_ref[...], sa_ref[0, :], sha_ref[0, :])
    o_ref[...] = jnp.dot(
        xm.astype(jnp.bfloat16), w_ref[...], preferred_element_type=jnp.float32
    ).astype(jnp.bfloat16)


def _qkv(x2, sa, sha, wqkv):
    return pl.pallas_call(
        _qkv_body,
        grid=(S // BQ,),
        in_specs=[
            pl.BlockSpec((BQ, D), lambda i: (i, 0)),
            pl.BlockSpec((1, D), lambda i: (0, 0)),
            pl.BlockSpec((1, D), lambda i: (0, 0)),
            pl.BlockSpec((D, 3 * D), lambda i: (0, 0)),
        ],
        out_specs=pl.BlockSpec((BQ, 3 * D), lambda i: (i, 0)),
        out_shape=jax.ShapeDtypeStruct((S, 3 * D), jnp.bfloat16),
    )(x2, sa, sha, wqkv)


def _attn_body(q_ref, k_ref, v_ref, o_ref):
    s = (
        lax.dot_general(
            q_ref[...],
            k_ref[...],
            (((1,), (1,)), ((), ())),
            preferred_element_type=jnp.float32,
        )
        * SCALE
    )
    m = jnp.max(s, axis=-1, keepdims=True)
    p = jnp.exp(s - m)
    l = jnp.sum(p, axis=-1, keepdims=True)
    o = jnp.dot(
        p.astype(jnp.bfloat16), v_ref[...], preferred_element_type=jnp.float32
    )
    o_ref[...] = (o / l).astype(jnp.bfloat16)


def _attention(qkv):
    return pl.pallas_call(
        _attn_body,
        grid=(H_LOC, S // BQ),
        in_specs=[
            pl.BlockSpec((BQ, DH), lambda h, i: (i, h)),
            pl.BlockSpec((S, DH), lambda h, i: (0, H_LOC + h)),
            pl.BlockSpec((S, DH), lambda h, i: (0, 2 * H_LOC + h)),
        ],
        out_specs=pl.BlockSpec((BQ, DH), lambda h, i: (i, h)),
        out_shape=jax.ShapeDtypeStruct((S, D), jnp.bfloat16),
    )(qkv, qkv, qkv)


def _matmul_body(a_ref, w_ref, o_ref):
    o_ref[...] = jnp.dot(
        a_ref[...], w_ref[...], preferred_element_type=jnp.float32
    ).astype(jnp.bfloat16)


def _matmul(a, w):
    k = a.shape[1]
    return pl.pallas_call(
        _matmul_body,
        grid=(S // BQ,),
        in_specs=[
            pl.BlockSpec((BQ, k), lambda i: (i, 0)),
            pl.BlockSpec((k, D), lambda i: (0, 0)),
        ],
        out_specs=pl.BlockSpec((BQ, D), lambda i: (i, 0)),
        out_shape=jax.ShapeDtypeStruct((S, D), jnp.bfloat16),
    )(a, w)


def _ff_body(x_ref, sm_ref, shm_ref, w1_ref, w2_ref, o_ref):
    xm = _ln_mod(x_ref[...], sm_ref[0, :], shm_ref[0, :]).astype(jnp.bfloat16)
    h = jnp.dot(xm, w1_ref[...], preferred_element_type=jnp.float32)
    h = h * jax.nn.sigmoid(h)
    o_ref[...] = jnp.dot(
        h.astype(jnp.bfloat16), w2_ref[...], preferred_element_type=jnp.float32
    ).astype(jnp.bfloat16)


def _ffn(x1, sm, shm, w1, w2):
    dff = w1.shape[1]
    return pl.pallas_call(
        _ff_body,
        grid=(S // BQ,),
        in_specs=[
            pl.BlockSpec((BQ, D), lambda i: (i, 0)),
            pl.BlockSpec((1, D), lambda i: (0, 0)),
            pl.BlockSpec((1, D), lambda i: (0, 0)),
            pl.BlockSpec((D, dff), lambda i: (0, 0)),
            pl.BlockSpec((dff, D), lambda i: (0, 0)),
        ],
        out_specs=pl.BlockSpec((BQ, D), lambda i: (i, 0)),
        out_shape=jax.ShapeDtypeStruct((S, D), jnp.bfloat16),
    )(x1, sm, shm, w1, w2)


CH2 = S // (2 * N_DEV)

_RING = [0, 4, 8, 12, 13, 9, 5, 1, 2, 6, 10, 14, 15, 11, 7, 3]
_RIDX = [0] * N_DEV
_NEXT = [0] * N_DEV
_PREV = [0] * N_DEV
for _i, _p in enumerate(_RING):
    _RIDX[_p] = _i
    _NEXT[_p] = _RING[(_i + 1) % N_DEV]
    _PREV[_p] = _RING[(_i - 1) % N_DEV]


def _ar_body(partial_hbm, res_hbm, gate_ref, nbr_ref, out_hbm,
             commA, commB, pbufA, pbufB, rbufA, rbufB,
             sendA, recvA, sendB, recvB, creditA, creditB, local_sem,
             storeA_sems, storeB_sems):
    r = nbr_ref[0]
    left = nbr_ref[1]
    right = nbr_ref[2]
    gate = gate_ref[0, :]
    f32 = jnp.float32

    barrier = pltpu.get_barrier_semaphore()
    for nbr in (left, right):
        pl.semaphore_signal(
            barrier, inc=1, device_id=(nbr,),
            device_id_type=pl.DeviceIdType.MESH,
        )
    pl.semaphore_wait(barrier, 2)

    def rowsA(idx):
        return pl.ds(idx * CH2, CH2)

    def rowsB(idx):
        return pl.ds(S // 2 + idx * CH2, CH2)

    def copy(src, dst):
        cp = pltpu.make_async_copy(src, dst, local_sem)
        cp.start()
        cp.wait()

    copy(partial_hbm.at[rowsA(r)], commA.at[0])
    copy(partial_hbm.at[rowsB(r)], commB.at[0])

    def store_out(rbuf, sl, sems, rows, cidx, summed_f32):
        rbuf[sl, :, :] = rbuf[sl, :, :] + gate * summed_f32
        pltpu.make_async_copy(
            rbuf.at[sl], out_hbm.at[rows(cidx)], sems.at[sl]
        ).start()

    def store_wait(rbuf, sl, sems):
        pltpu.make_async_copy(
            rbuf.at[sl], out_hbm.at[rowsA(0)], sems.at[sl]
        ).wait()

    for s in range(2 * N_DEV - 2):
        src_slot = s % 2
        dst_slot = (s + 1) % 2
        if s > 0:
            pl.semaphore_wait(creditA, 1)
            pl.semaphore_wait(creditB, 1)
        rdmaA = pltpu.make_async_remote_copy(
            src_ref=commA.at[src_slot],
            dst_ref=commA.at[dst_slot],
            send_sem=sendA.at[src_slot],
            recv_sem=recvA.at[dst_slot],
            device_id=(right,),
            device_id_type=pl.DeviceIdType.MESH,
        )
        rdmaB = pltpu.make_async_remote_copy(
            src_ref=commB.at[src_slot],
            dst_ref=commB.at[dst_slot],
            send_sem=sendB.at[src_slot],
            recv_sem=recvB.at[dst_slot],
            device_id=(left,),
            device_id_type=pl.DeviceIdType.MESH,
        )
        rdmaA.start()
        rdmaB.start()
        sl = s % 2
        if s < N_DEV - 1:
            ciA = lax.rem(r - s - 1 + 2 * N_DEV, N_DEV)
            ciB = lax.rem(r + s + 1, N_DEV)
            copy(partial_hbm.at[rowsA(ciA)], pbufA)
            copy(partial_hbm.at[rowsB(ciB)], pbufB)
            if s == N_DEV - 2:
                copy(res_hbm.at[rowsA(ciA)], rbufA.at[sl])
                copy(res_hbm.at[rowsB(ciB)], rbufB.at[sl])
            rdmaA.wait_send()
            rdmaA.wait_recv()
            accA = commA[dst_slot, :, :].astype(f32) + pbufA[...].astype(f32)
            commA[dst_slot, :, :] = accA.astype(jnp.bfloat16)
            rdmaB.wait_send()
            rdmaB.wait_recv()
            accB = commB[dst_slot, :, :].astype(f32) + pbufB[...].astype(f32)
            commB[dst_slot, :, :] = accB.astype(jnp.bfloat16)
            if s == N_DEV - 2:
                store_out(rbufA, sl, storeA_sems, rowsA, ciA, accA)
                store_out(rbufB, sl, storeB_sems, rowsB, ciB, accB)
        else:
            t = s - (N_DEV - 1)
            ciA = lax.rem(r - t + 2 * N_DEV, N_DEV)
            ciB = lax.rem(r + t, N_DEV)
            if s >= N_DEV:
                store_wait(rbufA, sl, storeA_sems)
                store_wait(rbufB, sl, storeB_sems)
            copy(res_hbm.at[rowsA(ciA)], rbufA.at[sl])
            copy(res_hbm.at[rowsB(ciB)], rbufB.at[sl])
            rdmaA.wait_send()
            rdmaA.wait_recv()
            store_out(rbufA, sl, storeA_sems, rowsA, ciA,
                      commA[dst_slot, :, :].astype(f32))
            rdmaB.wait_send()
            rdmaB.wait_recv()
            store_out(rbufB, sl, storeB_sems, rowsB, ciB,
                      commB[dst_slot, :, :].astype(f32))
        if s < 2 * N_DEV - 3:
            pl.semaphore_signal(
                creditA, inc=1, device_id=(left,),
                device_id_type=pl.DeviceIdType.MESH,
            )
            pl.semaphore_signal(
                creditB, inc=1, device_id=(right,),
                device_id_type=pl.DeviceIdType.MESH,
            )
    for sl in (0, 1):
        store_wait(rbufA, sl, storeA_sems)
        store_wait(rbufB, sl, storeB_sems)


def _allreduce_residual(partial, res, gate, nbrs, collective_id):
    return pl.pallas_call(
        _ar_body,
        in_specs=[
            pl.BlockSpec(memory_space=pl.ANY),
            pl.BlockSpec(memory_space=pl.ANY),
            pl.BlockSpec(memory_space=pltpu.VMEM),
            pl.BlockSpec(memory_space=pltpu.SMEM),
        ],
        out_specs=pl.BlockSpec(memory_space=pl.ANY),
        out_shape=jax.ShapeDtypeStruct((S, D), jnp.float32),
        scratch_shapes=[
            pltpu.VMEM((2, CH2, D), jnp.bfloat16),
            pltpu.VMEM((2, CH2, D), jnp.bfloat16),
            pltpu.VMEM((CH2, D), jnp.bfloat16),
            pltpu.VMEM((CH2, D), jnp.bfloat16),
            pltpu.VMEM((2, CH2, D), jnp.float32),
            pltpu.VMEM((2, CH2, D), jnp.float32),
            pltpu.SemaphoreType.DMA((2,)),
            pltpu.SemaphoreType.DMA((2,)),
            pltpu.SemaphoreType.DMA((2,)),
            pltpu.SemaphoreType.DMA((2,)),
            pltpu.SemaphoreType.REGULAR,
            pltpu.SemaphoreType.REGULAR,
            pltpu.SemaphoreType.DMA,
            pltpu.SemaphoreType.DMA((2,)),
            pltpu.SemaphoreType.DMA((2,)),
        ],
        compiler_params=_CompilerParams(collective_id=collective_id),
    )(partial, res, gate, nbrs)


def kernel(x, Wq, Wk, Wv, Wo, t_emb, W_mod, W_ff1, W_ff2):
    x2 = x.reshape(S, D)
    mod = t_emb @ W_mod
    sa, sha, ga, sm, shm, gm = jnp.split(mod, 6, axis=-1)

    pos = lax.axis_index("i")
    nbrs = jnp.stack([
        jnp.asarray(_RIDX, jnp.int32)[pos],
        jnp.asarray(_PREV, jnp.int32)[pos],
        jnp.asarray(_NEXT, jnp.int32)[pos],
    ])

    wqkv = jnp.concatenate([Wq, Wk, Wv], axis=1).astype(jnp.bfloat16)
    qkv = _qkv(x2, sa, sha, wqkv)
    attn = _attention(qkv)
    part1 = _matmul(attn, Wo.astype(jnp.bfloat16))
    x1 = _allreduce_residual(part1, x2, ga, nbrs, collective_id=0)

    part2 = _ffn(x1, sm, shm, W_ff1.astype(jnp.bfloat16),
                 W_ff2.astype(jnp.bfloat16))
    out = _allreduce_residual(part2, x1, gm, nbrs, collective_id=1)
    return out.reshape(1, S, D)
